# baseline (speedup 1.0000x reference)
import jax
import jax.numpy as jnp
from jax.experimental import pallas as pl

_F32 = jnp.float32
_BF16 = jnp.bfloat16


def _chamfer_body(at_ref, bt_ref, a16_ref, b16_ref, out_ref):
    at = at_ref[0]
    bt = bt_ref[0]
    a16 = a16_ref[0]
    b16 = b16_ref[0]
    b2 = jnp.sum(bt * bt, axis=0, keepdims=True)
    p1 = b2.astype(_BF16)
    r1 = b2 - p1.astype(_F32)
    p2 = r1.astype(_BF16)
    p3 = (r1 - p2.astype(_F32)).astype(_BF16)
    row = jax.lax.broadcasted_iota(jnp.int32, b16.shape, 0)
    b16 = jnp.where(row == 9, jnp.broadcast_to(p1, b16.shape), b16)
    b16 = jnp.where(row == 10, jnp.broadcast_to(p2, b16.shape), b16)
    b16 = jnp.where(row == 11, jnp.broadcast_to(p3, b16.shape), b16)
    d = jax.lax.dot_general(
        a16, b16, (((0,), (0,)), ((), ())),
        preferred_element_type=_F32)
    m = jnp.min(d, axis=1)
    a2 = jnp.sum(at * at, axis=0)
    loss = jnp.mean(a2 + m)
    out_ref[...] = jnp.broadcast_to(loss, out_ref.shape)


def kernel(adv_pc, ori_pc):
    B, K, _ = adv_pc.shape
    N = ori_pc.shape[1]
    at = jnp.pad(adv_pc, ((0, 0), (0, 0), (0, 5))).transpose(0, 2, 1)
    bt = jnp.pad(ori_pc, ((0, 0), (0, 0), (0, 5))).transpose(0, 2, 1)
    c = -2.0 * at[:, :3, :]
    c_hi = c.astype(_BF16)
    c_lo = (c - c_hi.astype(_F32)).astype(_BF16)
    ones = jnp.ones((B, 3, K), dtype=_BF16)
    zeros4 = jnp.zeros((B, 4, K), dtype=_BF16)
    a16 = jnp.concatenate([c_hi, c_lo, c_hi, ones, zeros4], axis=1)
    bq = bt[:, :3, :]
    b_hi = bq.astype(_BF16)
    b_lo = (bq - b_hi.astype(_F32)).astype(_BF16)
    zeros7 = jnp.zeros((B, 7, N), dtype=_BF16)
    b16 = jnp.concatenate([b_hi, b_hi, b_lo, zeros7], axis=1)
    out = pl.pallas_call(
        _chamfer_body,
        grid=(B,),
        in_specs=[
            pl.BlockSpec((1, 8, K), lambda b: (b, 0, 0)),
            pl.BlockSpec((1, 8, N), lambda b: (b, 0, 0)),
            pl.BlockSpec((1, 16, K), lambda b: (b, 0, 0)),
            pl.BlockSpec((1, 16, N), lambda b: (b, 0, 0)),
        ],
        out_specs=pl.BlockSpec((1, 1, 128), lambda b: (b, 0, 0)),
        out_shape=jax.ShapeDtypeStruct((B, 1, 128), jnp.float32),
    )(at, bt, a16, b16)
    return jnp.mean(out[:, 0, 0])


# speedup vs baseline: 1.3096x; 1.3096x over previous
import jax
import jax.numpy as jnp
from jax.experimental import pallas as pl

_F32 = jnp.float32
_BF16 = jnp.bfloat16
_CHUNKS = 4


def _chamfer_body(at_ref, bt_ref, out_ref):
    at = at_ref[0]
    bt = bt_ref[0]
    K = at.shape[1]
    c = -2.0 * at[:3, :]
    c_hi = c.astype(_BF16)
    c_lo = (c - c_hi.astype(_F32)).astype(_BF16)
    a16 = jnp.concatenate(
        [c_hi, c_lo, c_hi,
         jnp.ones((3, K), _BF16), jnp.zeros((4, K), _BF16)], axis=0)
    b = bt[:3, :]
    b_hi = b.astype(_BF16)
    b_lo = (b - b_hi.astype(_F32)).astype(_BF16)
    b2 = jnp.sum(bt * bt, axis=0, keepdims=True)
    p1 = b2.astype(_BF16)
    r1 = b2 - p1.astype(_F32)
    p2 = r1.astype(_BF16)
    p3 = (r1 - p2.astype(_F32)).astype(_BF16)
    b16 = jnp.concatenate(
        [b_hi, b_hi, b_lo, p1, p2, p3,
         jnp.zeros((4, bt.shape[1]), _BF16)], axis=0)
    a2 = jnp.sum(at * at, axis=0)
    kc = K // _CHUNKS
    total = None
    for i in range(_CHUNKS):
        ai = a16[:, i * kc:(i + 1) * kc]
        di = jax.lax.dot_general(
            ai, b16, (((0,), (0,)), ((), ())),
            preferred_element_type=_F32)
        mi = jnp.min(di, axis=1)
        si = jnp.sum(a2[i * kc:(i + 1) * kc] + mi)
        total = si if total is None else total + si
    loss = total / K
    out_ref[...] = jnp.broadcast_to(loss, out_ref.shape)


def kernel(adv_pc, ori_pc):
    B, K, _ = adv_pc.shape
    N = ori_pc.shape[1]
    at = jnp.pad(adv_pc, ((0, 0), (0, 0), (0, 5))).transpose(0, 2, 1)
    bt = jnp.pad(ori_pc, ((0, 0), (0, 0), (0, 5))).transpose(0, 2, 1)
    out = pl.pallas_call(
        _chamfer_body,
        grid=(B,),
        in_specs=[
            pl.BlockSpec((1, 8, K), lambda b: (b, 0, 0)),
            pl.BlockSpec((1, 8, N), lambda b: (b, 0, 0)),
        ],
        out_specs=pl.BlockSpec((1, 1, 128), lambda b: (b, 0, 0)),
        out_shape=jax.ShapeDtypeStruct((B, 1, 128), jnp.float32),
    )(at, bt)
    return jnp.mean(out[:, 0, 0])


# speedup vs baseline: 1.3642x; 1.0417x over previous
import jax
import jax.numpy as jnp
from jax.experimental import pallas as pl

_BPS = 2


def _chamfer_body(p_ref, out_ref):
    for j in range(_BPS):
        at = p_ref[j, :8, :]
        bt = p_ref[j, 8:, :]
        row_a = jax.lax.broadcasted_iota(jnp.int32, at.shape, 0)
        a_aug = jnp.where(row_a == 3, 1.0, -2.0 * at)
        b2 = jnp.sum(bt * bt, axis=0, keepdims=True)
        row_b = jax.lax.broadcasted_iota(jnp.int32, bt.shape, 0)
        bt_aug = jnp.where(row_b == 3, b2, bt)
        d = jax.lax.dot_general(
            a_aug, bt_aug, (((0,), (0,)), ((), ())),
            preferred_element_type=jnp.float32)
        m = jnp.min(d, axis=1)
        a2 = jnp.sum(at * at, axis=0)
        loss = jnp.mean(a2 + m)
        total = loss if j == 0 else total + loss
    out_ref[...] = jnp.broadcast_to(total, out_ref.shape)


def kernel(adv_pc, ori_pc):
    B, K, _ = adv_pc.shape
    N = ori_pc.shape[1]
    pts = jnp.concatenate([adv_pc, ori_pc], axis=2)
    p = jnp.pad(pts, ((0, 0), (0, 0), (0, 2)))
    p = p.transpose(0, 2, 1)
    p = jnp.concatenate(
        [p[:, :3], jnp.zeros((B, 5, K), jnp.float32),
         p[:, 3:6], jnp.zeros((B, 5, K), jnp.float32)], axis=1)
    steps = B // _BPS
    out = pl.pallas_call(
        _chamfer_body,
        grid=(steps,),
        in_specs=[pl.BlockSpec((_BPS, 16, K), lambda b: (b, 0, 0))],
        out_specs=pl.BlockSpec((1, 1, 128), lambda b: (b, 0, 0)),
        out_shape=jax.ShapeDtypeStruct((steps, 1, 128), jnp.float32),
    )(p)
    return jnp.sum(out[:, 0, 0]) / B


# speedup vs baseline: 1.4755x; 1.0816x over previous
import jax
import jax.numpy as jnp
from jax.experimental import pallas as pl

_BPS = 4


def _chamfer_body(p_ref, out_ref):
    for j in range(_BPS):
        at = p_ref[j, :4, :]
        bt = p_ref[j, 4:, :]
        row_a = jax.lax.broadcasted_iota(jnp.int32, at.shape, 0)
        a_aug = jnp.where(row_a == 3, 1.0, -2.0 * at)
        b2 = jnp.sum(bt * bt, axis=0, keepdims=True)
        row_b = jax.lax.broadcasted_iota(jnp.int32, bt.shape, 0)
        bt_aug = jnp.where(row_b == 3, b2, bt)
        d = jax.lax.dot_general(
            a_aug, bt_aug, (((0,), (0,)), ((), ())),
            preferred_element_type=jnp.float32)
        m = jnp.min(d, axis=1)
        a2 = jnp.sum(at * at, axis=0)
        loss = jnp.mean(a2 + m)
        total = loss if j == 0 else total + loss
    out_ref[...] = jnp.broadcast_to(total, out_ref.shape)


def kernel(adv_pc, ori_pc):
    B, K, _ = adv_pc.shape
    pts = jnp.concatenate(
        [adv_pc, jnp.zeros((B, K, 1), jnp.float32),
         ori_pc, jnp.zeros((B, K, 1), jnp.float32)], axis=2)
    p = pts.transpose(0, 2, 1)
    steps = B // _BPS
    out = pl.pallas_call(
        _chamfer_body,
        grid=(steps,),
        in_specs=[pl.BlockSpec((_BPS, 8, K), lambda b: (b, 0, 0))],
        out_specs=pl.BlockSpec((1, 1, 128), lambda b: (b, 0, 0)),
        out_shape=jax.ShapeDtypeStruct((steps, 1, 128), jnp.float32),
    )(p)
    return jnp.sum(out[:, 0, 0]) / B
